# Initial kernel scaffold; baseline (speedup 1.0000x reference)
import jax
import jax.numpy as jnp
from jax.experimental import pallas as pl


def kernel(p_init, r_matrix, indices_neigh_tri, W1a, b1a, W2a, b2a, W1b, b1b, W2b, b2b, W1c, b1c, W2c, b2c):
    raise NotImplementedError("write your pallas kernel here")


# speedup vs baseline: 49.4623x; 49.4623x over previous
import functools

import jax
import jax.numpy as jnp
from jax import lax
from jax.experimental import pallas as pl
from jax.experimental.pallas import tpu as pltpu
from jax.experimental.pallas import tpu_sc as plsc

N = 50000
K = 16
DR = 3
RD = K * DR
H = 128
NW = 32
CHUNK = 25024
NPAD = NW * CHUNK
BT = 2000
GRID = N // BT


def _sc_gather(f, idx_pad):
    mesh = plsc.VectorSubcoreMesh(core_axis_name="c", subcore_axis_name="s")

    @functools.partial(
        pl.kernel,
        out_type=jax.ShapeDtypeStruct((NPAD,), jnp.float32),
        mesh=mesh,
        scratch_types=[
            pltpu.VMEM((N,), jnp.float32),
            pltpu.VMEM((CHUNK,), jnp.int32),
            pltpu.VMEM((CHUNK,), jnp.float32),
        ],
        compiler_params=pltpu.CompilerParams(needs_layout_passes=False),
    )
    def gather_kernel(f_hbm, idx_hbm, out_hbm, f_v, idx_v, out_v):
        wid = lax.axis_index("s") * 2 + lax.axis_index("c")
        base = wid * CHUNK
        pltpu.sync_copy(f_hbm, f_v)
        pltpu.sync_copy(idx_hbm.at[pl.ds(base, CHUNK)], idx_v)

        def body(j, carry):
            o = j * 16
            iv = idx_v[pl.ds(o, 16)]
            out_v[pl.ds(o, 16)] = plsc.load_gather(f_v, [iv])
            return carry

        lax.fori_loop(0, CHUNK // 16, body, 0)
        pltpu.sync_copy(out_v, out_hbm.at[pl.ds(base, CHUNK)])

    return gather_kernel(f, idx_pad)


def _tc_block_body(r_ref, fn_ref, f_ref, w1r_ref, w1d_ref, b1_ref, w2_ref,
                   b2_ref, out_ref):
    diff = f_ref[...] - fn_ref[...]
    acc = jnp.dot(r_ref[...], w1r_ref[...], preferred_element_type=jnp.float32)
    acc += jnp.dot(diff, w1d_ref[...], preferred_element_type=jnp.float32)
    h = jnp.maximum(acc + b1_ref[...], 0.0)
    out_ref[...] = jnp.sum(h * w2_ref[...], axis=1, keepdims=True) + b2_ref[...]


def _tc_block(r_flat, fn, f_col, w1r, w1d, b1, w2row, b2):
    return pl.pallas_call(
        _tc_block_body,
        grid=(GRID,),
        in_specs=[
            pl.BlockSpec((BT, RD), lambda i: (i, 0)),
            pl.BlockSpec((BT, K), lambda i: (i, 0)),
            pl.BlockSpec((BT, 1), lambda i: (i, 0)),
            pl.BlockSpec((RD, H), lambda i: (0, 0)),
            pl.BlockSpec((K, H), lambda i: (0, 0)),
            pl.BlockSpec((1, H), lambda i: (0, 0)),
            pl.BlockSpec((1, H), lambda i: (0, 0)),
            pl.BlockSpec((1, 1), lambda i: (0, 0)),
        ],
        out_specs=pl.BlockSpec((BT, 1), lambda i: (i, 0)),
        out_shape=jax.ShapeDtypeStruct((N, 1), jnp.float32),
    )(r_flat, fn, f_col, w1r, w1d, b1, w2row, b2)


def _softmax_body(x_ref, o_ref):
    x = x_ref[...]
    m = jnp.max(x)
    e = jnp.exp(x - m)
    o_ref[...] = e / jnp.sum(e)


def _softmax(x2d):
    return pl.pallas_call(
        _softmax_body,
        out_shape=jax.ShapeDtypeStruct(x2d.shape, jnp.float32),
    )(x2d)


def kernel(p_init, r_matrix, indices_neigh_tri,
           W1a, b1a, W2a, b2a,
           W1b, b1b, W2b, b2b,
           W1c, b1c, W2c, b2c):
    neigh = indices_neigh_tri[:, 1:]
    idx_pad = jnp.concatenate(
        [neigh.reshape(-1), jnp.zeros((NPAD - N * K,), jnp.int32)])
    r_flat = r_matrix.reshape(N, RD)
    rsel = (4 * jnp.arange(K)[:, None] + jnp.arange(DR)[None, :]).reshape(-1)

    f = p_init
    for W1, b1, W2, b2 in ((W1a, b1a, W2a, b2a),
                           (W1b, b1b, W2b, b2b),
                           (W1c, b1c, W2c, b2c)):
        w1r = W1[rsel]
        w1d = W1[DR::DR + 1]
        fn = _sc_gather(f, idx_pad)[:N * K].reshape(N, K)
        f = _tc_block(r_flat, fn, f.reshape(N, 1), w1r, w1d,
                      b1.reshape(1, H), W2.reshape(1, H),
                      b2.reshape(1, 1))[:, 0]

    return _softmax(f.reshape(50, 1000)).reshape(N)


# speedup vs baseline: 98.2398x; 1.9862x over previous
import functools

import jax
import jax.numpy as jnp
from jax import lax
from jax.experimental import pallas as pl
from jax.experimental.pallas import tpu as pltpu
from jax.experimental.pallas import tpu_sc as plsc

N = 50000
K = 16
DR = 3
RD = K * DR
H = 128
NW = 32
NP = 53248
CW = NP // NW
BT = 2048
GRID = -(-N // BT)


def _sc_gather(f_row, idx_t):
    mesh = plsc.VectorSubcoreMesh(core_axis_name="c", subcore_axis_name="s")

    @functools.partial(
        pl.kernel,
        out_type=jax.ShapeDtypeStruct((K, NP), jnp.float32),
        mesh=mesh,
        scratch_types=[
            pltpu.VMEM((N,), jnp.float32),
            pltpu.VMEM((K, CW), jnp.int32),
            pltpu.VMEM((K, CW), jnp.float32),
        ],
        compiler_params=pltpu.CompilerParams(needs_layout_passes=False),
    )
    def gather_kernel(f_hbm, idx_hbm, out_hbm, f_v, idx_v, out_v):
        wid = lax.axis_index("s") * 2 + lax.axis_index("c")
        c0 = wid * CW
        pltpu.sync_copy(f_hbm.at[0], f_v)
        pltpu.sync_copy(idx_hbm.at[:, pl.ds(c0, CW)], idx_v)

        for k in range(K):
            def body(j, carry, k=k):
                o = j * 16
                iv = idx_v[k, pl.ds(o, 16)]
                out_v[k, pl.ds(o, 16)] = plsc.load_gather(f_v, [iv])
                return carry

            lax.fori_loop(0, CW // 16, body, 0)

        pltpu.sync_copy(out_v, out_hbm.at[:, pl.ds(c0, CW)])

    return gather_kernel(f_row, idx_t)


def _tc_block_body(rt_ref, fnt_ref, ft_ref, w1rt_ref, w1dt_ref, b1_ref,
                   w2_ref, b2_ref, out_ref):
    diff_t = ft_ref[...] - fnt_ref[...]
    acc = jnp.dot(w1rt_ref[...], rt_ref[...],
                  preferred_element_type=jnp.float32)
    acc += jnp.dot(w1dt_ref[...], diff_t,
                   preferred_element_type=jnp.float32)
    h = jnp.maximum(acc + b1_ref[...], 0.0)
    out_ref[...] = (jnp.sum(h * w2_ref[...], axis=0, keepdims=True)
                    + b2_ref[...])


def _tc_block(r_t, fn_t, f_row, w1rt, w1dt, b1_col, w2_col, b2):
    return pl.pallas_call(
        _tc_block_body,
        grid=(GRID,),
        in_specs=[
            pl.BlockSpec((RD, BT), lambda i: (0, i)),
            pl.BlockSpec((K, BT), lambda i: (0, i)),
            pl.BlockSpec((1, BT), lambda i: (0, i)),
            pl.BlockSpec((H, RD), lambda i: (0, 0)),
            pl.BlockSpec((H, K), lambda i: (0, 0)),
            pl.BlockSpec((H, 1), lambda i: (0, 0)),
            pl.BlockSpec((H, 1), lambda i: (0, 0)),
            pl.BlockSpec((1, 1), lambda i: (0, 0)),
        ],
        out_specs=pl.BlockSpec((1, BT), lambda i: (0, i)),
        out_shape=jax.ShapeDtypeStruct((1, N), jnp.float32),
    )(r_t, fn_t, f_row, w1rt, w1dt, b1_col, w2_col, b2)


def _softmax_body(x_ref, o_ref):
    x = x_ref[...]
    m = jnp.max(x)
    e = jnp.exp(x - m)
    o_ref[...] = e / jnp.sum(e)


def _softmax(x_row):
    return pl.pallas_call(
        _softmax_body,
        out_shape=jax.ShapeDtypeStruct((1, N), jnp.float32),
    )(x_row)


def kernel(p_init, r_matrix, indices_neigh_tri,
           W1a, b1a, W2a, b2a,
           W1b, b1b, W2b, b2b,
           W1c, b1c, W2c, b2c):
    neigh = indices_neigh_tri[:, 1:]
    idx_t = jnp.pad(neigh.T, ((0, 0), (0, NP - N)))
    r_t = r_matrix.reshape(N, RD).T
    rsel = (4 * jnp.arange(K)[:, None] + jnp.arange(DR)[None, :]).reshape(-1)

    f_row = p_init.reshape(1, N)
    for W1, b1, W2, b2 in ((W1a, b1a, W2a, b2a),
                           (W1b, b1b, W2b, b2b),
                           (W1c, b1c, W2c, b2c)):
        w1rt = W1[rsel].T
        w1dt = W1[DR::DR + 1].T
        fn_t = _sc_gather(f_row, idx_t)
        f_row = _tc_block(r_t, fn_t, f_row, w1rt, w1dt,
                          b1.reshape(H, 1), W2.reshape(H, 1),
                          b2.reshape(1, 1))

    return _softmax(f_row).reshape(N)
